# baseline (speedup 1.0000x reference)
import functools

import jax
import jax.numpy as jnp
from jax import lax
from jax.experimental import pallas as pl
from jax.experimental.pallas import tpu as pltpu
from jax.experimental.pallas import tpu_sc as plsc

BATCH = 16384
EMBED_DIM = 32
NUM_WORKERS = 32
B_PER_W = BATCH // NUM_WORKERS
CHUNK = 128
N_CHUNKS = B_PER_W // CHUNK
N_GROUPS = B_PER_W // 16


def _sc_body(uidx_hbm, iidx_hbm, uembT_hbm, iembT_hbm, ubias_hbm, ibias_hbm,
             out_hbm,
             uidx_v, iidx_v, ut_v, it_v, ub_v, ib_v, out_v,
             sem_u, sem_i, sem_ub, sem_ib):
    wid = lax.axis_index("s") * 2 + lax.axis_index("c")
    row0 = wid * N_CHUNKS
    base = wid * B_PER_W

    pltpu.sync_copy(uidx_hbm.at[pl.ds(row0, N_CHUNKS)], uidx_v)
    pltpu.sync_copy(iidx_hbm.at[pl.ds(row0, N_CHUNKS)], iidx_v)

    def copies():
        cs = []
        for j in range(N_CHUNKS):
            dst = pl.ds(j * CHUNK, CHUNK)
            for d in range(EMBED_DIM):
                cs.append(pltpu.make_async_copy(
                    uembT_hbm.at[d].at[uidx_v.at[j]], ut_v.at[d, dst], sem_u))
                cs.append(pltpu.make_async_copy(
                    iembT_hbm.at[d].at[iidx_v.at[j]], it_v.at[d, dst], sem_i))
            cs.append(pltpu.make_async_copy(
                ubias_hbm.at[uidx_v.at[j]], ub_v.at[dst], sem_ub))
            cs.append(pltpu.make_async_copy(
                ibias_hbm.at[iidx_v.at[j]], ib_v.at[dst], sem_ib))
        return cs

    for c in copies():
        c.start()
    for c in copies():
        c.wait()

    def grp_body(g, carry):
        off = pl.multiple_of(g * 16, 16)
        sl = pl.ds(off, 16)
        acc = ub_v[sl] + ib_v[sl]
        for d in range(EMBED_DIM):
            acc = acc + ut_v[d, sl] * it_v[d, sl]
        out_v[sl] = 1.0 / (1.0 + jnp.exp(-acc))
        return carry

    lax.fori_loop(0, N_GROUPS, grp_body, 0, unroll=2)

    pltpu.sync_copy(out_v, out_hbm.at[pl.ds(base, B_PER_W)])


@jax.jit
def _mf_sc(uidx, iidx, uembT, iembT, ubias, ibias):
    mesh = plsc.VectorSubcoreMesh(core_axis_name="c", subcore_axis_name="s")
    f = functools.partial(
        pl.kernel,
        mesh=mesh,
        compiler_params=pltpu.CompilerParams(
            needs_layout_passes=False, use_tc_tiling_on_sc=False),
        out_type=jax.ShapeDtypeStruct((BATCH,), jnp.float32),
        scratch_types=[
            pltpu.VMEM((N_CHUNKS, CHUNK), jnp.int32),
            pltpu.VMEM((N_CHUNKS, CHUNK), jnp.int32),
            pltpu.VMEM((EMBED_DIM, B_PER_W), jnp.float32),
            pltpu.VMEM((EMBED_DIM, B_PER_W), jnp.float32),
            pltpu.VMEM((B_PER_W,), jnp.float32),
            pltpu.VMEM((B_PER_W,), jnp.float32),
            pltpu.VMEM((B_PER_W,), jnp.float32),
            pltpu.SemaphoreType.DMA,
            pltpu.SemaphoreType.DMA,
            pltpu.SemaphoreType.DMA,
            pltpu.SemaphoreType.DMA,
        ],
    )(_sc_body)
    return f(uidx, iidx, uembT, iembT, ubias, ibias)


def kernel(user_idx, item_idx, user_emb, item_emb, user_bias, item_bias):
    uidx = user_idx.astype(jnp.int32).reshape(BATCH // CHUNK, CHUNK)
    iidx = item_idx.astype(jnp.int32).reshape(BATCH // CHUNK, CHUNK)
    uembT = jnp.swapaxes(user_emb, 0, 1)
    iembT = jnp.swapaxes(item_emb, 0, 1)
    ubias = user_bias.reshape(-1)
    ibias = item_bias.reshape(-1)
    return _mf_sc(uidx, iidx, uembT, iembT, ubias, ibias)


# speedup vs baseline: 5.7736x; 5.7736x over previous
import functools

import jax
import jax.numpy as jnp
from jax import lax
from jax.experimental import pallas as pl
from jax.experimental.pallas import tpu as pltpu
from jax.experimental.pallas import tpu_sc as plsc

BATCH = 16384
EMBED_DIM = 32
NUM_WORKERS = 32
B_PER_W = BATCH // NUM_WORKERS
CHUNK = 128
N_CHUNKS = B_PER_W // CHUNK


def _sc_body(uidx_hbm, iidx_hbm, uemb_hbm, iemb_hbm, ubias_hbm, ibias_hbm,
             out_hbm,
             uidx_v, iidx_v, urows_v, irows_v, ub_v, ib_v, work_v, work2_v,
             dot_v, out_v, sem):
    wid = lax.axis_index("s") * 2 + lax.axis_index("c")
    row0 = wid * N_CHUNKS
    base = wid * B_PER_W

    pltpu.sync_copy(uidx_hbm.at[pl.ds(row0, N_CHUNKS)], uidx_v)
    pltpu.sync_copy(iidx_hbm.at[pl.ds(row0, N_CHUNKS)], iidx_v)

    copies = []
    for j in range(N_CHUNKS):
        dst = pl.ds(j * CHUNK, CHUNK)
        copies.append(pltpu.make_async_copy(
            uemb_hbm.at[uidx_v.at[j]], urows_v.at[dst], sem))
        copies.append(pltpu.make_async_copy(
            iemb_hbm.at[iidx_v.at[j]], irows_v.at[dst], sem))
        copies.append(pltpu.make_async_copy(
            ubias_hbm.at[uidx_v.at[j]], ub_v.at[dst], sem))
        copies.append(pltpu.make_async_copy(
            ibias_hbm.at[iidx_v.at[j]], ib_v.at[dst], sem))
    for c in copies:
        c.start()
    for c in copies:
        c.wait()

    def dot_body(b, carry):
        u0 = urows_v[b, pl.ds(0, 16)]
        u1 = urows_v[b, pl.ds(16, 16)]
        v0 = irows_v[b, pl.ds(0, 16)]
        v1 = irows_v[b, pl.ds(16, 16)]
        off = pl.multiple_of(b * 16, 16)
        work_v[pl.ds(off, 16)] = u0 * v0 + u1 * v1
        return carry

    lax.fori_loop(0, B_PER_W, dot_body, 0, unroll=4)

    iota = lax.iota(jnp.int32, 16)

    def fold(src, dst, n_out):
        def body(i, carry):
            src_base = i * 32
            a = plsc.load_gather(src, [src_base + iota * 2])
            b = plsc.load_gather(src, [src_base + iota * 2 + 1])
            off = pl.multiple_of(i * 16, 16)
            dst[pl.ds(off, 16)] = a + b
            return carry

        lax.fori_loop(0, n_out // 16, body, 0, unroll=4)

    fold(work_v, work2_v, 4096)
    fold(work2_v, work_v, 2048)
    fold(work_v, work2_v, 1024)
    fold(work2_v, dot_v, 512)

    def sig_body(g, carry):
        off = pl.multiple_of(g * 16, 16)
        x = dot_v[pl.ds(off, 16)] + ub_v[pl.ds(off, 16)] + ib_v[pl.ds(off, 16)]
        out_v[pl.ds(off, 16)] = 1.0 / (1.0 + jnp.exp(-x))
        return carry

    lax.fori_loop(0, B_PER_W // 16, sig_body, 0, unroll=4)

    pltpu.sync_copy(out_v, out_hbm.at[pl.ds(base, B_PER_W)])


@jax.jit
def _mf_sc(uidx, iidx, uemb, iemb, ubias, ibias):
    mesh = plsc.VectorSubcoreMesh(core_axis_name="c", subcore_axis_name="s")
    f = functools.partial(
        pl.kernel,
        mesh=mesh,
        compiler_params=pltpu.CompilerParams(
            needs_layout_passes=False, use_tc_tiling_on_sc=False),
        out_type=jax.ShapeDtypeStruct((BATCH,), jnp.float32),
        scratch_types=[
            pltpu.VMEM((N_CHUNKS, CHUNK), jnp.int32),
            pltpu.VMEM((N_CHUNKS, CHUNK), jnp.int32),
            pltpu.VMEM((B_PER_W, EMBED_DIM), jnp.float32),
            pltpu.VMEM((B_PER_W, EMBED_DIM), jnp.float32),
            pltpu.VMEM((B_PER_W,), jnp.float32),
            pltpu.VMEM((B_PER_W,), jnp.float32),
            pltpu.VMEM((B_PER_W * 16,), jnp.float32),
            pltpu.VMEM((B_PER_W * 8,), jnp.float32),
            pltpu.VMEM((B_PER_W,), jnp.float32),
            pltpu.VMEM((B_PER_W,), jnp.float32),
            pltpu.SemaphoreType.DMA,
        ],
    )(_sc_body)
    return f(uidx, iidx, uemb, iemb, ubias, ibias)


def kernel(user_idx, item_idx, user_emb, item_emb, user_bias, item_bias):
    uidx = user_idx.astype(jnp.int32).reshape(BATCH // CHUNK, CHUNK)
    iidx = item_idx.astype(jnp.int32).reshape(BATCH // CHUNK, CHUNK)
    uflat = lax.optimization_barrier(user_emb.reshape(-1))
    iflat = lax.optimization_barrier(item_emb.reshape(-1))
    uemb = uflat.reshape(N_USERS_SHAPE)
    iemb = iflat.reshape(N_USERS_SHAPE)
    ubias = user_bias.reshape(-1)
    ibias = item_bias.reshape(-1)
    return _mf_sc(uidx, iidx, uemb, iemb, ubias, ibias)


N_USERS_SHAPE = (1000000, EMBED_DIM)


# speedup vs baseline: 14.4386x; 2.5008x over previous
import functools

import jax
import jax.numpy as jnp
from jax import lax
from jax.experimental import pallas as pl
from jax.experimental.pallas import tpu as pltpu
from jax.experimental.pallas import tpu_sc as plsc

BATCH = 16384
EMBED_DIM = 32
N_USERS = 1000000
NUM_WORKERS = 32
B_PER_W = BATCH // NUM_WORKERS
CHUNK = 128
N_CHUNKS = B_PER_W // CHUNK
BIAS_ROWS = 7816
WAVE = 8
N_WAVES = B_PER_W // WAVE


def _sc_body(uidx_hbm, iidx_hbm, uembT_hbm, iembT_hbm, ubias_hbm, ibias_hbm,
             out_hbm,
             uidx_v, iidx_v, cols_v, brow_v, brow_idx_v,
             ub_v, ib_v, work_v, work2_v, dot_v, out_v, sem):
    wid = lax.axis_index("s") * 2 + lax.axis_index("c")
    base = wid * B_PER_W

    pltpu.sync_copy(uidx_hbm.at[pl.ds(base, B_PER_W)],
                    uidx_v.at[pl.ds(0, B_PER_W)])
    pltpu.sync_copy(iidx_hbm.at[pl.ds(base, B_PER_W)],
                    iidx_v.at[pl.ds(0, B_PER_W)])

    iota = lax.iota(jnp.int32, 16)
    zero16 = iota * 0

    def wave_body(w, carry):
        off8 = pl.multiple_of(w * WAVE, 8)
        uvec = uidx_v[pl.ds(off8, 16)]
        ivec = iidx_v[pl.ds(off8, 16)]
        copies = []
        for l in range(WAVE):
            u = uvec[l]
            i = ivec[l]
            cbu = pl.multiple_of((u >> 7) * CHUNK, CHUNK)
            cbi = pl.multiple_of((i >> 7) * CHUNK, CHUNK)
            copies.append(pltpu.make_async_copy(
                uembT_hbm.at[:, pl.ds(cbu, CHUNK)], cols_v.at[0, l], sem))
            copies.append(pltpu.make_async_copy(
                iembT_hbm.at[:, pl.ds(cbi, CHUNK)], cols_v.at[1, l], sem))
        for c in copies:
            c.start()
        for c in copies:
            c.wait()
        for l in range(WAVE):
            cu = zero16 + (uvec[l] & 127)
            ci = zero16 + (ivec[l] & 127)
            u_lo = plsc.load_gather(cols_v, [zero16, zero16 + l, iota, cu])
            u_hi = plsc.load_gather(cols_v, [zero16, zero16 + l, iota + 16,
                                             cu])
            v_lo = plsc.load_gather(cols_v, [zero16 + 1, zero16 + l, iota,
                                             ci])
            v_hi = plsc.load_gather(cols_v, [zero16 + 1, zero16 + l, iota + 16,
                                             ci])
            poff = pl.multiple_of((w * WAVE + l) * 16, 16)
            work_v[pl.ds(poff, 16)] = u_lo * v_lo + u_hi * v_hi
        return carry

    lax.fori_loop(0, N_WAVES, wave_body, 0)

    def bias_for_table(bias_hbm, idx_v, dst_v):
        for j in range(N_CHUNKS):
            for g in range(CHUNK // 16):
                v = idx_v[pl.ds(j * CHUNK + g * 16, 16)]
                brow_idx_v[pl.ds(g * 16, 16)] = v >> 7
            pltpu.make_async_copy(
                bias_hbm.at[brow_idx_v], brow_v, sem).start()
            pltpu.make_async_copy(
                bias_hbm.at[brow_idx_v], brow_v, sem).wait()
            for g in range(CHUNK // 16):
                v = idx_v[pl.ds(j * CHUNK + g * 16, 16)]
                rowv = g * 16 + iota
                dst_v[pl.ds(j * CHUNK + g * 16, 16)] = plsc.load_gather(
                    brow_v, [rowv, v & 127])

    bias_for_table(ubias_hbm, uidx_v, ub_v)
    bias_for_table(ibias_hbm, iidx_v, ib_v)

    def fold(src, dst, n_out):
        def body(i, carry):
            src_base = i * 32
            a = plsc.load_gather(src, [src_base + iota * 2])
            b = plsc.load_gather(src, [src_base + iota * 2 + 1])
            dst[pl.ds(pl.multiple_of(i * 16, 16), 16)] = a + b
            return carry

        lax.fori_loop(0, n_out // 16, body, 0, unroll=4)

    fold(work_v, work2_v, 4096)
    fold(work2_v, work_v, 2048)
    fold(work_v, work2_v, 1024)
    fold(work2_v, dot_v, 512)

    def sig_body(g, carry):
        off = pl.multiple_of(g * 16, 16)
        x = dot_v[pl.ds(off, 16)] + ub_v[pl.ds(off, 16)] + ib_v[pl.ds(off, 16)]
        out_v[pl.ds(off, 16)] = 1.0 / (1.0 + jnp.exp(-x))
        return carry

    lax.fori_loop(0, B_PER_W // 16, sig_body, 0, unroll=4)

    pltpu.sync_copy(out_v, out_hbm.at[pl.ds(base, B_PER_W)])


@jax.jit
def _mf_sc(uidx, iidx, uembT, iembT, ubias2d, ibias2d):
    mesh = plsc.VectorSubcoreMesh(core_axis_name="c", subcore_axis_name="s")
    f = functools.partial(
        pl.kernel,
        mesh=mesh,
        compiler_params=pltpu.CompilerParams(
            needs_layout_passes=False, use_tc_tiling_on_sc=True),
        out_type=jax.ShapeDtypeStruct((BATCH,), jnp.float32),
        scratch_types=[
            pltpu.VMEM((B_PER_W + 16,), jnp.int32),
            pltpu.VMEM((B_PER_W + 16,), jnp.int32),
            pltpu.VMEM((2, WAVE, EMBED_DIM, CHUNK), jnp.float32),
            pltpu.VMEM((CHUNK, CHUNK), jnp.float32),
            pltpu.VMEM((CHUNK,), jnp.int32),
            pltpu.VMEM((B_PER_W,), jnp.float32),
            pltpu.VMEM((B_PER_W,), jnp.float32),
            pltpu.VMEM((B_PER_W * 16,), jnp.float32),
            pltpu.VMEM((B_PER_W * 8,), jnp.float32),
            pltpu.VMEM((B_PER_W,), jnp.float32),
            pltpu.VMEM((B_PER_W,), jnp.float32),
            pltpu.SemaphoreType.DMA,
        ],
    )(_sc_body)
    return f(uidx, iidx, uembT, iembT, ubias2d, ibias2d)


def kernel(user_idx, item_idx, user_emb, item_emb, user_bias, item_bias):
    uidx = user_idx.astype(jnp.int32)
    iidx = item_idx.astype(jnp.int32)
    uembT = jnp.swapaxes(user_emb, 0, 1)
    iembT = jnp.swapaxes(item_emb, 0, 1)
    pad = jnp.zeros((BIAS_ROWS * CHUNK - N_USERS,), jnp.float32)
    ubias2d = jnp.concatenate([user_bias.reshape(-1), pad]).reshape(
        BIAS_ROWS, CHUNK)
    ibias2d = jnp.concatenate([item_bias.reshape(-1), pad]).reshape(
        BIAS_ROWS, CHUNK)
    return _mf_sc(uidx, iidx, uembT, iembT, ubias2d, ibias2d)


# speedup vs baseline: 14.8223x; 1.0266x over previous
import functools

import jax
import jax.numpy as jnp
from jax import lax
from jax.experimental import pallas as pl
from jax.experimental.pallas import tpu as pltpu
from jax.experimental.pallas import tpu_sc as plsc

BATCH = 16384
EMBED_DIM = 32
N_USERS = 1000000
NUM_WORKERS = 32
B_PER_W = BATCH // NUM_WORKERS
CHUNK = 128
N_CHUNKS = B_PER_W // CHUNK
BIAS_ROWS = 7816
WAVE = 4
N_WAVES = B_PER_W // WAVE


def _sc_body(uidx_hbm, iidx_hbm, uembT_hbm, iembT_hbm, ubias_hbm, ibias_hbm,
             out_hbm,
             uidx_v, iidx_v, cols_v, brow_v, brow_idx_v,
             ub_v, ib_v, work_v, work2_v, dot_v, out_v, sem_a, sem_b):
    wid = lax.axis_index("s") * 2 + lax.axis_index("c")
    base = wid * B_PER_W

    pltpu.sync_copy(uidx_hbm.at[pl.ds(base, B_PER_W)],
                    uidx_v.at[pl.ds(0, B_PER_W)])
    pltpu.sync_copy(iidx_hbm.at[pl.ds(base, B_PER_W)],
                    iidx_v.at[pl.ds(0, B_PER_W)])

    iota = lax.iota(jnp.int32, 16)
    zero16 = iota * 0

    def fetch_wave(q, parity, buf, wsem):
        off8 = pl.multiple_of(q * 8, 8)
        lane0 = parity * WAVE
        uvec = uidx_v[pl.ds(off8, 16)]
        ivec = iidx_v[pl.ds(off8, 16)]
        copies = []
        for l in range(WAVE):
            u = uvec[lane0 + l]
            i = ivec[lane0 + l]
            cbu = pl.multiple_of((u >> 7) * CHUNK, CHUNK)
            cbi = pl.multiple_of((i >> 7) * CHUNK, CHUNK)
            copies.append(pltpu.make_async_copy(
                uembT_hbm.at[:, pl.ds(cbu, CHUNK)], cols_v.at[buf, 0, l],
                wsem))
            copies.append(pltpu.make_async_copy(
                iembT_hbm.at[:, pl.ds(cbi, CHUNK)], cols_v.at[buf, 1, l],
                wsem))
        for c in copies:
            c.start()

    def drain_wave(buf, wsem):
        for l in range(WAVE):
            pltpu.make_async_copy(
                uembT_hbm.at[:, pl.ds(0, CHUNK)], cols_v.at[buf, 0, l],
                wsem).wait()
            pltpu.make_async_copy(
                uembT_hbm.at[:, pl.ds(0, CHUNK)], cols_v.at[buf, 1, l],
                wsem).wait()

    def extract_wave(q, parity, buf):
        off8 = pl.multiple_of(q * 8, 8)
        lane0 = parity * WAVE
        uvec = uidx_v[pl.ds(off8, 16)]
        ivec = iidx_v[pl.ds(off8, 16)]
        for l in range(WAVE):
            cu = zero16 + (uvec[lane0 + l] & 127)
            ci = zero16 + (ivec[lane0 + l] & 127)
            u_lo = plsc.load_gather(cols_v, [zero16 + buf, zero16, zero16 + l,
                                             iota, cu])
            u_hi = plsc.load_gather(cols_v, [zero16 + buf, zero16, zero16 + l,
                                             iota + 16, cu])
            v_lo = plsc.load_gather(cols_v, [zero16 + buf, zero16 + 1,
                                             zero16 + l, iota, ci])
            v_hi = plsc.load_gather(cols_v, [zero16 + buf, zero16 + 1,
                                             zero16 + l, iota + 16, ci])
            poff = pl.multiple_of((q * 8 + parity * WAVE + l) * 16, 16)
            work_v[pl.ds(poff, 16)] = u_lo * v_lo + u_hi * v_hi

    def pair_body(p, carry):
        fetch_wave(p, 1, 1, sem_b)
        drain_wave(0, sem_a)
        extract_wave(p, 0, 0)

        @pl.when(p < N_WAVES // 2 - 1)
        def _():
            fetch_wave(p + 1, 0, 0, sem_a)

        drain_wave(1, sem_b)
        extract_wave(p, 1, 1)
        return carry

    fetch_wave(0, 0, 0, sem_a)
    lax.fori_loop(0, N_WAVES // 2, pair_body, 0)

    def bias_for_table(bias_hbm, idx_v, dst_v):
        for j in range(N_CHUNKS):
            for g in range(CHUNK // 16):
                v = idx_v[pl.ds(j * CHUNK + g * 16, 16)]
                brow_idx_v[pl.ds(g * 16, 16)] = v >> 7
            pltpu.make_async_copy(
                bias_hbm.at[brow_idx_v], brow_v, sem_a).start()
            pltpu.make_async_copy(
                bias_hbm.at[brow_idx_v], brow_v, sem_a).wait()
            for g in range(CHUNK // 16):
                v = idx_v[pl.ds(j * CHUNK + g * 16, 16)]
                rowv = g * 16 + iota
                dst_v[pl.ds(j * CHUNK + g * 16, 16)] = plsc.load_gather(
                    brow_v, [rowv, v & 127])

    bias_for_table(ubias_hbm, uidx_v, ub_v)
    bias_for_table(ibias_hbm, iidx_v, ib_v)

    def fold(src, dst, n_out):
        def body(i, carry):
            src_base = i * 32
            a = plsc.load_gather(src, [src_base + iota * 2])
            b = plsc.load_gather(src, [src_base + iota * 2 + 1])
            dst[pl.ds(pl.multiple_of(i * 16, 16), 16)] = a + b
            return carry

        lax.fori_loop(0, n_out // 16, body, 0, unroll=4)

    fold(work_v, work2_v, 4096)
    fold(work2_v, work_v, 2048)
    fold(work_v, work2_v, 1024)
    fold(work2_v, dot_v, 512)

    def sig_body(g, carry):
        off = pl.multiple_of(g * 16, 16)
        x = dot_v[pl.ds(off, 16)] + ub_v[pl.ds(off, 16)] + ib_v[pl.ds(off, 16)]
        out_v[pl.ds(off, 16)] = 1.0 / (1.0 + jnp.exp(-x))
        return carry

    lax.fori_loop(0, B_PER_W // 16, sig_body, 0, unroll=4)

    pltpu.sync_copy(out_v, out_hbm.at[pl.ds(base, B_PER_W)])


@jax.jit
def _mf_sc(uidx, iidx, uembT, iembT, ubias2d, ibias2d):
    mesh = plsc.VectorSubcoreMesh(core_axis_name="c", subcore_axis_name="s")
    f = functools.partial(
        pl.kernel,
        mesh=mesh,
        compiler_params=pltpu.CompilerParams(
            needs_layout_passes=False, use_tc_tiling_on_sc=True),
        out_type=jax.ShapeDtypeStruct((BATCH,), jnp.float32),
        scratch_types=[
            pltpu.VMEM((B_PER_W + 16,), jnp.int32),
            pltpu.VMEM((B_PER_W + 16,), jnp.int32),
            pltpu.VMEM((2, 2, WAVE, EMBED_DIM, CHUNK), jnp.float32),
            pltpu.VMEM((CHUNK, CHUNK), jnp.float32),
            pltpu.VMEM((CHUNK,), jnp.int32),
            pltpu.VMEM((B_PER_W,), jnp.float32),
            pltpu.VMEM((B_PER_W,), jnp.float32),
            pltpu.VMEM((B_PER_W * 16,), jnp.float32),
            pltpu.VMEM((B_PER_W * 8,), jnp.float32),
            pltpu.VMEM((B_PER_W,), jnp.float32),
            pltpu.VMEM((B_PER_W,), jnp.float32),
            pltpu.SemaphoreType.DMA,
            pltpu.SemaphoreType.DMA,
        ],
    )(_sc_body)
    return f(uidx, iidx, uembT, iembT, ubias2d, ibias2d)


def kernel(user_idx, item_idx, user_emb, item_emb, user_bias, item_bias):
    uidx = user_idx.astype(jnp.int32)
    iidx = item_idx.astype(jnp.int32)
    uembT = jnp.swapaxes(user_emb, 0, 1)
    iembT = jnp.swapaxes(item_emb, 0, 1)
    pad = jnp.zeros((BIAS_ROWS * CHUNK - N_USERS,), jnp.float32)
    ubias2d = jnp.concatenate([user_bias.reshape(-1), pad]).reshape(
        BIAS_ROWS, CHUNK)
    ibias2d = jnp.concatenate([item_bias.reshape(-1), pad]).reshape(
        BIAS_ROWS, CHUNK)
    return _mf_sc(uidx, iidx, uembT, iembT, ubias2d, ibias2d)
